# baseline (speedup 1.0000x reference)
import functools

import jax
import jax.numpy as jnp
from jax import lax
from jax.experimental import pallas as pl
from jax.experimental.pallas import tpu as pltpu
from jax.experimental.pallas import tpu_sc as plsc

B, H, I, E, K = 4096, 2048, 1024, 64, 8
BLK_B = 512
TILE = 256
NT = (B * K + E * (TILE - 1) + TILE - 1) // TILE
C = NT * TILE
NW = 32
TPW = B // NW


def _router_body(x_ref, gw_ref, ids_ref, rank_ref, w_ref, offs_ref, te_ref,
                 tv_ref, carry):
    b = pl.program_id(0)

    @pl.when(b == 0)
    def _():
        carry[...] = jnp.zeros_like(carry)

    x = x_ref[...]
    logits = lax.dot_general(x, gw_ref[...], (((1,), (1,)), ((), ())),
                             preferred_element_type=jnp.float32)
    iota = lax.broadcasted_iota(jnp.int32, (BLK_B, E), 1)
    l = logits
    onehots, vals, idxs = [], [], []
    for _ in range(K):
        mx = jnp.max(l, axis=1, keepdims=True)
        idx = jnp.min(jnp.where(l == mx, iota, E), axis=1, keepdims=True)
        oh = (iota == idx)
        onehots.append(oh)
        vals.append(mx)
        idxs.append(idx)
        l = jnp.where(oh, -jnp.inf, l)
    v0 = vals[0]
    exps = [jnp.exp(v - v0) for v in vals]
    denom = exps[0]
    for ev in exps[1:]:
        denom = denom + ev

    sel = onehots[0].astype(jnp.float32)
    for oh in onehots[1:]:
        sel = sel + oh.astype(jnp.float32)

    ri = lax.broadcasted_iota(jnp.int32, (BLK_B, BLK_B), 0)
    ci = lax.broadcasted_iota(jnp.int32, (BLK_B, BLK_B), 1)
    tri = (ri > ci).astype(jnp.bfloat16)
    cum = lax.dot_general(tri, sel.astype(jnp.bfloat16),
                          (((1,), (0,)), ((), ())),
                          preferred_element_type=jnp.float32)
    posf = cum + carry[0:1, :]

    rank_cols, id_cols, w_cols = [], [], []
    for k in range(K):
        oh = onehots[k]
        rank_cols.append(jnp.sum(jnp.where(oh, posf, 0.0), axis=1,
                                 keepdims=True))
        id_cols.append(idxs[k])
        w_cols.append(exps[k] / denom)
    ids_ref[...] = jnp.concatenate(id_cols, axis=1)
    rank_ref[...] = jnp.concatenate(rank_cols, axis=1).astype(jnp.int32)
    w_ref[...] = jnp.concatenate(w_cols, axis=1)

    newc = carry[0:1, :] + jnp.sum(sel, axis=0, keepdims=True)
    carry[0:1, :] = newc

    p = jnp.floor((newc + (TILE - 1.0)) * (1.0 / TILE)) * TILE
    rie = lax.broadcasted_iota(jnp.int32, (E, E), 0)
    cie = lax.broadcasted_iota(jnp.int32, (E, E), 1)
    triu = (rie < cie).astype(jnp.bfloat16)
    offs = lax.dot_general(p.astype(jnp.bfloat16), triu,
                           (((1,), (0,)), ((), ())),
                           preferred_element_type=jnp.float32)
    offs_ref[...] = offs.astype(jnp.int32)
    ends = (offs + p) * (1.0 / TILE)
    tt = jnp.sum(p) * (1.0 / TILE)
    ti = lax.broadcasted_iota(jnp.int32, (NT, E), 0).astype(jnp.float32)
    te = jnp.sum((ti >= ends).astype(jnp.float32), axis=1, keepdims=True)
    te_ref[...] = jnp.minimum(te, E - 1).astype(jnp.int32)
    tiv = lax.broadcasted_iota(jnp.int32, (NT, 1), 0).astype(jnp.float32)
    tv_ref[...] = (tiv < tt).astype(jnp.int32)


def _router(hidden_states, gate_weight):
    return pl.pallas_call(
        _router_body,
        grid=(B // BLK_B,),
        in_specs=[
            pl.BlockSpec((BLK_B, H), lambda b: (b, 0)),
            pl.BlockSpec((E, H), lambda b: (0, 0)),
        ],
        out_specs=[
            pl.BlockSpec((BLK_B, K), lambda b: (b, 0)),
            pl.BlockSpec((BLK_B, K), lambda b: (b, 0)),
            pl.BlockSpec((BLK_B, K), lambda b: (b, 0)),
            pl.BlockSpec((1, E), lambda b: (0, 0)),
            pl.BlockSpec((NT, 1), lambda b: (0, 0)),
            pl.BlockSpec((NT, 1), lambda b: (0, 0)),
        ],
        out_shape=[
            jax.ShapeDtypeStruct((B, K), jnp.int32),
            jax.ShapeDtypeStruct((B, K), jnp.int32),
            jax.ShapeDtypeStruct((B, K), jnp.float32),
            jax.ShapeDtypeStruct((1, E), jnp.int32),
            jax.ShapeDtypeStruct((NT, 1), jnp.int32),
            jax.ShapeDtypeStruct((NT, 1), jnp.int32),
        ],
        scratch_shapes=[pltpu.VMEM((8, E), jnp.float32)],
    )(hidden_states, gate_weight)


def _dispatch_body(x_hbm, idsT_hbm, rankT_hbm, wT_hbm, offs_hbm,
                   xs_hbm, ws_hbm, stok_hbm,
                   offs_v, ids_v, rank_v, wv_v, slots_v, stok_v, xbuf_v):
    cid = lax.axis_index("c")
    sid = lax.axis_index("s")
    wid = sid * 2 + cid
    base = wid * TPW

    pltpu.sync_copy(offs_hbm, offs_v)
    pltpu.sync_copy(idsT_hbm.at[:, pl.ds(base, TPW)], ids_v)
    pltpu.sync_copy(rankT_hbm.at[:, pl.ds(base, TPW)], rank_v)
    pltpu.sync_copy(wT_hbm.at[:, pl.ds(base, TPW)], wv_v)

    iota = lax.iota(jnp.int32, 16)
    for k in range(K):
        for j in range(TPW // 16):
            e = ids_v[k, pl.ds(j * 16, 16)]
            r = rank_v[k, pl.ds(j * 16, 16)]
            slot = plsc.load_gather(offs_v, [e]) + r
            slots_v[k, pl.ds(j * 16, 16)] = slot
            plsc.store_scatter(stok_v, [(j * 16 + iota) * K + k], slot)
    pltpu.sync_copy(stok_v, stok_hbm.at[pl.ds(base * K, TPW * K)])

    for c in range(TPW // 16):
        pltpu.sync_copy(x_hbm.at[pl.ds(base + c * 16, 16)], xbuf_v)
        for k in range(K):
            idx = slots_v[k, pl.ds(c * 16, 16)]
            pltpu.sync_copy(xbuf_v, xs_hbm.at[idx])
            pltpu.sync_copy(wv_v.at[k, pl.ds(c * 16, 16)], ws_hbm.at[idx])


def _dispatch(x, idsT, rankT, wT, offs):
    mesh = plsc.VectorSubcoreMesh(core_axis_name="c", subcore_axis_name="s")
    f = functools.partial(
        pl.kernel, _dispatch_body, mesh=mesh,
        out_type=[
            jax.ShapeDtypeStruct((C, H), jnp.float32),
            jax.ShapeDtypeStruct((C,), jnp.float32),
            jax.ShapeDtypeStruct((B * K,), jnp.int32),
        ],
        scratch_types=[
            pltpu.VMEM((E,), jnp.int32),
            pltpu.VMEM((K, TPW), jnp.int32),
            pltpu.VMEM((K, TPW), jnp.int32),
            pltpu.VMEM((K, TPW), jnp.float32),
            pltpu.VMEM((K, TPW), jnp.int32),
            pltpu.VMEM((TPW * K,), jnp.int32),
            pltpu.VMEM((16, H), jnp.float32),
        ],
        compiler_params=pltpu.CompilerParams(needs_layout_passes=False),
    )()
    return f(x, idsT, rankT, wT, offs)


def _gmm_body(te_ref, tv_ref, xs_ref, wg_ref, wu_ref, wd_ref, w_ref, out_ref,
              g_scr, h_scr):
    i = pl.program_id(0)
    s = pl.program_id(1)

    @pl.when((tv_ref[i] == 1) & (s == 0))
    def _():
        g_scr[...] = lax.dot_general(xs_ref[...], wg_ref[0],
                                     (((1,), (1,)), ((), ())),
                                     preferred_element_type=jnp.float32)

    @pl.when((tv_ref[i] == 1) & (s == 1))
    def _():
        u = lax.dot_general(xs_ref[...], wu_ref[0], (((1,), (1,)), ((), ())),
                            preferred_element_type=jnp.float32)
        g = g_scr[...]
        h_scr[...] = g * jax.nn.sigmoid(g) * u

    @pl.when((tv_ref[i] == 1) & (s == 2))
    def _():
        eo = lax.dot_general(h_scr[...], wd_ref[0], (((1,), (1,)), ((), ())),
                             preferred_element_type=jnp.float32)
        out_ref[...] = eo * w_ref[...]


def _gmm(te, tv, xs, wg, wu, wd, ws2):
    def _xi(i, s, te, tv):
        return (jnp.where(tv[i] == 1, i, NT - 1), 0)

    def _wgi(i, s, te, tv):
        ii = jnp.minimum(i + 1, NT - 1)
        return (jnp.where(s >= 1, te[ii], te[i]), 0, 0)

    def _wui(i, s, te, tv):
        ii = jnp.minimum(i + 1, NT - 1)
        return (jnp.where(s >= 2, te[ii], te[i]), 0, 0)

    def _wdi(i, s, te, tv):
        return (te[i], 0, 0)

    grid_spec = pltpu.PrefetchScalarGridSpec(
        num_scalar_prefetch=2,
        grid=(NT, 3),
        in_specs=[
            pl.BlockSpec((TILE, H), _xi),
            pl.BlockSpec((1, I, H), _wgi),
            pl.BlockSpec((1, I, H), _wui),
            pl.BlockSpec((1, H, I), _wdi),
            pl.BlockSpec((TILE, 1), lambda i, s, te, tv: (i, 0)),
        ],
        out_specs=pl.BlockSpec((TILE, H), _xi),
        scratch_shapes=[
            pltpu.VMEM((TILE, I), jnp.float32),
            pltpu.VMEM((TILE, I), jnp.float32),
        ],
    )
    return pl.pallas_call(
        _gmm_body,
        grid_spec=grid_spec,
        out_shape=jax.ShapeDtypeStruct((C, H), jnp.float32),
        compiler_params=pltpu.CompilerParams(
            dimension_semantics=("arbitrary", "arbitrary"),
            vmem_limit_bytes=100 * 1024 * 1024,
        ),
    )(te, tv, xs, wg, wu, wd, ws2)


def _cgather_body(ys_hbm, stok_hbm, y8_hbm, stok_v, ybuf_v):
    cid = lax.axis_index("c")
    sid = lax.axis_index("s")
    wid = sid * 2 + cid
    base8 = wid * TPW * K

    pltpu.sync_copy(stok_hbm.at[pl.ds(base8, TPW * K)], stok_v)
    for j in range(TPW * K // 16):
        idx = stok_v[pl.ds(j * 16, 16)]
        pltpu.sync_copy(ys_hbm.at[idx], ybuf_v)
        pltpu.sync_copy(ybuf_v, y8_hbm.at[pl.ds(base8 + j * 16, 16)])


def _cgather(ys, stok):
    mesh = plsc.VectorSubcoreMesh(core_axis_name="c", subcore_axis_name="s")
    f = functools.partial(
        pl.kernel, _cgather_body, mesh=mesh,
        out_type=jax.ShapeDtypeStruct((B * K, H), jnp.float32),
        scratch_types=[
            pltpu.VMEM((TPW * K,), jnp.int32),
            pltpu.VMEM((16, H), jnp.float32),
        ],
        compiler_params=pltpu.CompilerParams(needs_layout_passes=False),
    )()
    return f(ys, stok)


RED_B = 128


def _reduce_body(y8_ref, out_ref):
    x = y8_ref[...].reshape(RED_B, K, H)
    out_ref[...] = jnp.sum(x, axis=1)


def _reduce(y8):
    return pl.pallas_call(
        _reduce_body,
        grid=(B // RED_B,),
        in_specs=[pl.BlockSpec((RED_B * K, H), lambda b: (b, 0))],
        out_specs=pl.BlockSpec((RED_B, H), lambda b: (b, 0)),
        out_shape=jax.ShapeDtypeStruct((B, H), jnp.float32),
    )(y8)


def kernel(hidden_states, gate_weight, w_gate_proj, w_up_proj, w_down_proj):
    ids, rank, w, offs, te, tv = _router(hidden_states, gate_weight)
    idsT = ids.T
    rankT = rank.T
    wT = w.T
    xs, ws, stok = _dispatch(hidden_states, idsT, rankT, wT,
                             offs.reshape(E))
    ys = _gmm(te.reshape(NT), tv.reshape(NT), xs, w_gate_proj, w_up_proj,
              w_down_proj, ws.reshape(C, 1))
    y8 = _cgather(ys, stok)
    return _reduce(y8)


# speedup vs baseline: 1.0956x; 1.0956x over previous
import functools

import jax
import jax.numpy as jnp
from jax import lax
from jax.experimental import pallas as pl
from jax.experimental.pallas import tpu as pltpu
from jax.experimental.pallas import tpu_sc as plsc

B, H, I, E, K = 4096, 2048, 1024, 64, 8
BLK_B = 512
TILE = 256
NT = (B * K + E * (TILE - 1) + TILE - 1) // TILE
C = NT * TILE
NW = 32
TPW = B // NW


def _router_body(x_ref, gw_ref, ids_ref, rank_ref, w_ref, offs_ref, te_ref,
                 tv_ref, carry):
    b = pl.program_id(0)

    @pl.when(b == 0)
    def _():
        carry[...] = jnp.zeros_like(carry)

    x = x_ref[...]
    logits = lax.dot_general(x, gw_ref[...], (((1,), (1,)), ((), ())),
                             preferred_element_type=jnp.float32)
    iota = lax.broadcasted_iota(jnp.int32, (BLK_B, E), 1)
    l = logits
    onehots, vals, idxs = [], [], []
    for _ in range(K):
        mx = jnp.max(l, axis=1, keepdims=True)
        idx = jnp.min(jnp.where(l == mx, iota, E), axis=1, keepdims=True)
        oh = (iota == idx)
        onehots.append(oh)
        vals.append(mx)
        idxs.append(idx)
        l = jnp.where(oh, -jnp.inf, l)
    v0 = vals[0]
    exps = [jnp.exp(v - v0) for v in vals]
    denom = exps[0]
    for ev in exps[1:]:
        denom = denom + ev

    sel = onehots[0].astype(jnp.float32)
    for oh in onehots[1:]:
        sel = sel + oh.astype(jnp.float32)

    ri = lax.broadcasted_iota(jnp.int32, (BLK_B, BLK_B), 0)
    ci = lax.broadcasted_iota(jnp.int32, (BLK_B, BLK_B), 1)
    tri = (ri > ci).astype(jnp.bfloat16)
    cum = lax.dot_general(tri, sel.astype(jnp.bfloat16),
                          (((1,), (0,)), ((), ())),
                          preferred_element_type=jnp.float32)
    posf = cum + carry[0:1, :]

    rank_cols, id_cols, w_cols = [], [], []
    for k in range(K):
        oh = onehots[k]
        rank_cols.append(jnp.sum(jnp.where(oh, posf, 0.0), axis=1,
                                 keepdims=True))
        id_cols.append(idxs[k])
        w_cols.append(exps[k] / denom)
    ids_ref[...] = jnp.concatenate(id_cols, axis=1)
    rank_ref[...] = jnp.concatenate(rank_cols, axis=1).astype(jnp.int32)
    w_ref[...] = jnp.concatenate(w_cols, axis=1)

    newc = carry[0:1, :] + jnp.sum(sel, axis=0, keepdims=True)
    carry[0:1, :] = newc

    p = jnp.floor((newc + (TILE - 1.0)) * (1.0 / TILE)) * TILE
    rie = lax.broadcasted_iota(jnp.int32, (E, E), 0)
    cie = lax.broadcasted_iota(jnp.int32, (E, E), 1)
    triu = (rie < cie).astype(jnp.bfloat16)
    offs = lax.dot_general(p.astype(jnp.bfloat16), triu,
                           (((1,), (0,)), ((), ())),
                           preferred_element_type=jnp.float32)
    offs_ref[...] = offs.astype(jnp.int32)
    ends = (offs + p) * (1.0 / TILE)
    tt = jnp.sum(p) * (1.0 / TILE)
    ti = lax.broadcasted_iota(jnp.int32, (NT, E), 0).astype(jnp.float32)
    te = jnp.sum((ti >= ends).astype(jnp.float32), axis=1, keepdims=True)
    te_ref[...] = jnp.minimum(te, E - 1).astype(jnp.int32)
    tiv = lax.broadcasted_iota(jnp.int32, (NT, 1), 0).astype(jnp.float32)
    tv_ref[...] = (tiv < tt).astype(jnp.int32)


def _router(hidden_states, gate_weight):
    return pl.pallas_call(
        _router_body,
        grid=(B // BLK_B,),
        in_specs=[
            pl.BlockSpec((BLK_B, H), lambda b: (b, 0)),
            pl.BlockSpec((E, H), lambda b: (0, 0)),
        ],
        out_specs=[
            pl.BlockSpec((BLK_B, K), lambda b: (b, 0)),
            pl.BlockSpec((BLK_B, K), lambda b: (b, 0)),
            pl.BlockSpec((BLK_B, K), lambda b: (b, 0)),
            pl.BlockSpec((1, E), lambda b: (0, 0)),
            pl.BlockSpec((NT, 1), lambda b: (0, 0)),
            pl.BlockSpec((NT, 1), lambda b: (0, 0)),
        ],
        out_shape=[
            jax.ShapeDtypeStruct((B, K), jnp.int32),
            jax.ShapeDtypeStruct((B, K), jnp.int32),
            jax.ShapeDtypeStruct((B, K), jnp.float32),
            jax.ShapeDtypeStruct((1, E), jnp.int32),
            jax.ShapeDtypeStruct((NT, 1), jnp.int32),
            jax.ShapeDtypeStruct((NT, 1), jnp.int32),
        ],
        scratch_shapes=[pltpu.VMEM((8, E), jnp.float32)],
    )(hidden_states, gate_weight)


def _dispatch_body(x_hbm, idsT_hbm, rankT_hbm, wT_hbm, offs_hbm,
                   xs_hbm, ws_hbm, stok_hbm,
                   offs_v, ids_v, rank_v, wv_v, slots_v, stok_v, xbuf_v):
    cid = lax.axis_index("c")
    sid = lax.axis_index("s")
    wid = sid * 2 + cid
    base = wid * TPW

    pltpu.sync_copy(offs_hbm, offs_v)
    pltpu.sync_copy(idsT_hbm.at[:, pl.ds(base, TPW)], ids_v)
    pltpu.sync_copy(rankT_hbm.at[:, pl.ds(base, TPW)], rank_v)
    pltpu.sync_copy(wT_hbm.at[:, pl.ds(base, TPW)], wv_v)

    iota = lax.iota(jnp.int32, 16)
    for k in range(K):
        for j in range(TPW // 16):
            e = ids_v[k, pl.ds(j * 16, 16)]
            r = rank_v[k, pl.ds(j * 16, 16)]
            slot = plsc.load_gather(offs_v, [e]) + r
            slots_v[k, pl.ds(j * 16, 16)] = slot
            plsc.store_scatter(stok_v, [(j * 16 + iota) * K + k], slot)
    pltpu.sync_copy(stok_v, stok_hbm.at[pl.ds(base * K, TPW * K)])

    for c in range(TPW // 16):
        pltpu.sync_copy(x_hbm.at[pl.ds(base + c * 16, 16)], xbuf_v)
        for k in range(K):
            idx = slots_v[k, pl.ds(c * 16, 16)]
            pltpu.sync_copy(xbuf_v, xs_hbm.at[idx])
            pltpu.sync_copy(wv_v.at[k, pl.ds(c * 16, 16)], ws_hbm.at[idx])


def _dispatch(x, idsT, rankT, wT, offs):
    mesh = plsc.VectorSubcoreMesh(core_axis_name="c", subcore_axis_name="s")
    f = functools.partial(
        pl.kernel, _dispatch_body, mesh=mesh,
        out_type=[
            jax.ShapeDtypeStruct((C, H), jnp.float32),
            jax.ShapeDtypeStruct((C,), jnp.float32),
            jax.ShapeDtypeStruct((B * K,), jnp.int32),
        ],
        scratch_types=[
            pltpu.VMEM((E,), jnp.int32),
            pltpu.VMEM((K, TPW), jnp.int32),
            pltpu.VMEM((K, TPW), jnp.int32),
            pltpu.VMEM((K, TPW), jnp.float32),
            pltpu.VMEM((K, TPW), jnp.int32),
            pltpu.VMEM((TPW * K,), jnp.int32),
            pltpu.VMEM((16, H), jnp.float32),
        ],
        compiler_params=pltpu.CompilerParams(needs_layout_passes=False),
    )()
    return f(x, idsT, rankT, wT, offs)


def _gmm_body(te_ref, tv_ref, xs_ref, wg_ref, wu_ref, wd_ref, w_ref, out_ref):
    i = pl.program_id(0)

    @pl.when(tv_ref[i] == 1)
    def _():
        x = xs_ref[...]
        g = lax.dot_general(x, wg_ref[0], (((1,), (1,)), ((), ())),
                            preferred_element_type=jnp.float32)
        u = lax.dot_general(x, wu_ref[0], (((1,), (1,)), ((), ())),
                            preferred_element_type=jnp.float32)
        h = g * jax.nn.sigmoid(g) * u
        eo = lax.dot_general(h, wd_ref[0], (((1,), (1,)), ((), ())),
                             preferred_element_type=jnp.float32)
        out_ref[...] = eo * w_ref[...]


def _gmm(te, tv, xs, wg, wu, wd, ws2):
    grid_spec = pltpu.PrefetchScalarGridSpec(
        num_scalar_prefetch=2,
        grid=(NT,),
        in_specs=[
            pl.BlockSpec((TILE, H),
                         lambda i, te, tv: (jnp.where(tv[i] == 1, i, NT - 1),
                                            0)),
            pl.BlockSpec((1, I, H), lambda i, te, tv: (te[i], 0, 0)),
            pl.BlockSpec((1, I, H), lambda i, te, tv: (te[i], 0, 0)),
            pl.BlockSpec((1, H, I), lambda i, te, tv: (te[i], 0, 0)),
            pl.BlockSpec((TILE, 1), lambda i, te, tv: (i, 0)),
        ],
        out_specs=pl.BlockSpec((TILE, H),
                               lambda i, te, tv: (jnp.where(tv[i] == 1, i,
                                                            NT - 1), 0)),
    )
    return pl.pallas_call(
        _gmm_body,
        grid_spec=grid_spec,
        out_shape=jax.ShapeDtypeStruct((C, H), jnp.float32),
        compiler_params=pltpu.CompilerParams(
            dimension_semantics=("arbitrary",),
            vmem_limit_bytes=100 * 1024 * 1024,
        ),
    )(te, tv, xs, wg, wu, wd, ws2)


def _cgather_body(ys_hbm, stok_hbm, y8_hbm, stok_v, ybuf_v):
    cid = lax.axis_index("c")
    sid = lax.axis_index("s")
    wid = sid * 2 + cid
    base8 = wid * TPW * K

    pltpu.sync_copy(stok_hbm.at[pl.ds(base8, TPW * K)], stok_v)
    for j in range(TPW * K // 32):
        idx = stok_v.at[pl.ds(j * 32, 32)]
        pltpu.sync_copy(ys_hbm.at[idx], ybuf_v)
        pltpu.sync_copy(ybuf_v, y8_hbm.at[pl.ds(base8 + j * 32, 32)])


def _cgather(ys, stok):
    mesh = plsc.VectorSubcoreMesh(core_axis_name="c", subcore_axis_name="s")
    f = functools.partial(
        pl.kernel, _cgather_body, mesh=mesh,
        out_type=jax.ShapeDtypeStruct((B * K, H), jnp.float32),
        scratch_types=[
            pltpu.VMEM((TPW * K,), jnp.int32),
            pltpu.VMEM((32, H), jnp.float32),
        ],
        compiler_params=pltpu.CompilerParams(needs_layout_passes=False),
    )()
    return f(ys, stok)


RED_B = 128


def _reduce_body(y8_ref, out_ref):
    x = y8_ref[...].reshape(RED_B, K, H)
    out_ref[...] = jnp.sum(x, axis=1)


def _reduce(y8):
    return pl.pallas_call(
        _reduce_body,
        grid=(B // RED_B,),
        in_specs=[pl.BlockSpec((RED_B * K, H), lambda b: (b, 0))],
        out_specs=pl.BlockSpec((RED_B, H), lambda b: (b, 0)),
        out_shape=jax.ShapeDtypeStruct((B, H), jnp.float32),
    )(y8)


def kernel(hidden_states, gate_weight, w_gate_proj, w_up_proj, w_down_proj):
    ids, rank, w, offs, te, tv = _router(hidden_states, gate_weight)
    idsT = ids.T
    rankT = rank.T
    wT = w.T
    xs, ws, stok = _dispatch(hidden_states, idsT, rankT, wT,
                             offs.reshape(E))
    ys = _gmm(te.reshape(NT), tv.reshape(NT), xs, w_gate_proj, w_up_proj,
              w_down_proj, ws.reshape(C, 1))
    y8 = _cgather(ys, stok)
    return _reduce(y8)


# speedup vs baseline: 1.0960x; 1.0004x over previous
import functools

import jax
import jax.numpy as jnp
from jax import lax
from jax.experimental import pallas as pl
from jax.experimental.pallas import tpu as pltpu
from jax.experimental.pallas import tpu_sc as plsc

B, H, I, E, K = 4096, 2048, 1024, 64, 8
BLK_B = 512
TILE = 256
NT = (B * K + E * (TILE - 1) + TILE - 1) // TILE
C = NT * TILE
NW = 32
TPW = B // NW


def _router_body(x_ref, gw_ref, ids_ref, rank_ref, w_ref, offs_ref, te_ref,
                 tv_ref, carry):
    b = pl.program_id(0)

    @pl.when(b == 0)
    def _():
        carry[...] = jnp.zeros_like(carry)

    x = x_ref[...]
    logits = lax.dot_general(x, gw_ref[...], (((1,), (1,)), ((), ())),
                             preferred_element_type=jnp.float32)
    iota = lax.broadcasted_iota(jnp.int32, (BLK_B, E), 1)
    l = logits
    onehots, vals, idxs = [], [], []
    for _ in range(K):
        mx = jnp.max(l, axis=1, keepdims=True)
        idx = jnp.min(jnp.where(l == mx, iota, E), axis=1, keepdims=True)
        oh = (iota == idx)
        onehots.append(oh)
        vals.append(mx)
        idxs.append(idx)
        l = jnp.where(oh, -jnp.inf, l)
    v0 = vals[0]
    exps = [jnp.exp(v - v0) for v in vals]
    denom = exps[0]
    for ev in exps[1:]:
        denom = denom + ev

    sel = onehots[0].astype(jnp.float32)
    for oh in onehots[1:]:
        sel = sel + oh.astype(jnp.float32)

    ri = lax.broadcasted_iota(jnp.int32, (BLK_B, BLK_B), 0)
    ci = lax.broadcasted_iota(jnp.int32, (BLK_B, BLK_B), 1)
    tri = (ri > ci).astype(jnp.bfloat16)
    cum = lax.dot_general(tri, sel.astype(jnp.bfloat16),
                          (((1,), (0,)), ((), ())),
                          preferred_element_type=jnp.float32)
    posf = cum + carry[0:1, :]

    rank_cols, id_cols, w_cols = [], [], []
    for k in range(K):
        oh = onehots[k]
        rank_cols.append(jnp.sum(jnp.where(oh, posf, 0.0), axis=1,
                                 keepdims=True))
        id_cols.append(idxs[k])
        w_cols.append(exps[k] / denom)
    ids_ref[...] = jnp.concatenate(id_cols, axis=1)
    rank_ref[...] = jnp.concatenate(rank_cols, axis=1).astype(jnp.int32)
    w_ref[...] = jnp.concatenate(w_cols, axis=1)

    newc = carry[0:1, :] + jnp.sum(sel, axis=0, keepdims=True)
    carry[0:1, :] = newc

    p = jnp.floor((newc + (TILE - 1.0)) * (1.0 / TILE)) * TILE
    rie = lax.broadcasted_iota(jnp.int32, (E, E), 0)
    cie = lax.broadcasted_iota(jnp.int32, (E, E), 1)
    triu = (rie < cie).astype(jnp.bfloat16)
    offs = lax.dot_general(p.astype(jnp.bfloat16), triu,
                           (((1,), (0,)), ((), ())),
                           preferred_element_type=jnp.float32)
    offs_ref[...] = offs.astype(jnp.int32)
    ends = (offs + p) * (1.0 / TILE)
    tt = jnp.sum(p) * (1.0 / TILE)
    ti = lax.broadcasted_iota(jnp.int32, (NT, E), 0).astype(jnp.float32)
    te = jnp.sum((ti >= ends).astype(jnp.float32), axis=1, keepdims=True)
    te_ref[...] = jnp.minimum(te, E - 1).astype(jnp.int32)
    tiv = lax.broadcasted_iota(jnp.int32, (NT, 1), 0).astype(jnp.float32)
    tv_ref[...] = (tiv < tt).astype(jnp.int32)


def _router(hidden_states, gate_weight):
    return pl.pallas_call(
        _router_body,
        grid=(B // BLK_B,),
        in_specs=[
            pl.BlockSpec((BLK_B, H), lambda b: (b, 0)),
            pl.BlockSpec((E, H), lambda b: (0, 0)),
        ],
        out_specs=[
            pl.BlockSpec((BLK_B, K), lambda b: (b, 0)),
            pl.BlockSpec((BLK_B, K), lambda b: (b, 0)),
            pl.BlockSpec((BLK_B, K), lambda b: (b, 0)),
            pl.BlockSpec((1, E), lambda b: (0, 0)),
            pl.BlockSpec((NT, 1), lambda b: (0, 0)),
            pl.BlockSpec((NT, 1), lambda b: (0, 0)),
        ],
        out_shape=[
            jax.ShapeDtypeStruct((B, K), jnp.int32),
            jax.ShapeDtypeStruct((B, K), jnp.int32),
            jax.ShapeDtypeStruct((B, K), jnp.float32),
            jax.ShapeDtypeStruct((1, E), jnp.int32),
            jax.ShapeDtypeStruct((NT, 1), jnp.int32),
            jax.ShapeDtypeStruct((NT, 1), jnp.int32),
        ],
        scratch_shapes=[pltpu.VMEM((8, E), jnp.float32)],
    )(hidden_states, gate_weight)


def _dispatch_body(x_hbm, idsT_hbm, rankT_hbm, wT_hbm, offs_hbm,
                   xs_hbm, ws_hbm, stok_hbm,
                   offs_v, ids_v, rank_v, wv_v, slots_v, stok_v, xbuf_v):
    cid = lax.axis_index("c")
    sid = lax.axis_index("s")
    wid = sid * 2 + cid
    base = wid * TPW

    pltpu.sync_copy(offs_hbm, offs_v)
    pltpu.sync_copy(idsT_hbm.at[:, pl.ds(base, TPW)], ids_v)
    pltpu.sync_copy(rankT_hbm.at[:, pl.ds(base, TPW)], rank_v)
    pltpu.sync_copy(wT_hbm.at[:, pl.ds(base, TPW)], wv_v)

    iota = lax.iota(jnp.int32, 16)
    for k in range(K):
        for j in range(TPW // 16):
            e = ids_v[k, pl.ds(j * 16, 16)]
            r = rank_v[k, pl.ds(j * 16, 16)]
            slot = plsc.load_gather(offs_v, [e]) + r
            slots_v[k * (TPW // 32) + j // 2, pl.ds((j % 2) * 16, 16)] = slot
            plsc.store_scatter(stok_v, [(j * 16 + iota) * K + k], slot)
    pltpu.sync_copy(stok_v, stok_hbm.at[pl.ds(base * K, TPW * K)])

    for c in range(TPW // 32):
        pltpu.sync_copy(x_hbm.at[pl.ds(base + c * 32, 32)], xbuf_v)
        for k in range(K):
            idx = slots_v.at[k * (TPW // 32) + c]
            pltpu.sync_copy(xbuf_v, xs_hbm.at[idx])
            pltpu.sync_copy(wv_v.at[k, pl.ds(c * 32, 32)], ws_hbm.at[idx])


def _dispatch(x, idsT, rankT, wT, offs):
    mesh = plsc.VectorSubcoreMesh(core_axis_name="c", subcore_axis_name="s")
    f = functools.partial(
        pl.kernel, _dispatch_body, mesh=mesh,
        out_type=[
            jax.ShapeDtypeStruct((C, H), jnp.float32),
            jax.ShapeDtypeStruct((C,), jnp.float32),
            jax.ShapeDtypeStruct((B * K,), jnp.int32),
        ],
        scratch_types=[
            pltpu.VMEM((E,), jnp.int32),
            pltpu.VMEM((K, TPW), jnp.int32),
            pltpu.VMEM((K, TPW), jnp.int32),
            pltpu.VMEM((K, TPW), jnp.float32),
            pltpu.VMEM((K * TPW // 32, 32), jnp.int32),
            pltpu.VMEM((TPW * K,), jnp.int32),
            pltpu.VMEM((32, H), jnp.float32),
        ],
        compiler_params=pltpu.CompilerParams(needs_layout_passes=False),
    )()
    return f(x, idsT, rankT, wT, offs)


def _gmm_body(te_ref, tv_ref, xs_ref, wg_ref, wu_ref, wd_ref, w_ref, out_ref):
    i = pl.program_id(0)

    @pl.when(tv_ref[i] == 1)
    def _():
        x = xs_ref[...]
        g = lax.dot_general(x, wg_ref[0], (((1,), (1,)), ((), ())),
                            preferred_element_type=jnp.float32)
        u = lax.dot_general(x, wu_ref[0], (((1,), (1,)), ((), ())),
                            preferred_element_type=jnp.float32)
        h = g * jax.nn.sigmoid(g) * u
        eo = lax.dot_general(h, wd_ref[0], (((1,), (1,)), ((), ())),
                             preferred_element_type=jnp.float32)
        out_ref[...] = eo * w_ref[...]


def _gmm(te, tv, xs, wg, wu, wd, ws2):
    grid_spec = pltpu.PrefetchScalarGridSpec(
        num_scalar_prefetch=2,
        grid=(NT,),
        in_specs=[
            pl.BlockSpec((TILE, H),
                         lambda i, te, tv: (jnp.where(tv[i] == 1, i, NT - 1),
                                            0)),
            pl.BlockSpec((1, I, H), lambda i, te, tv: (te[i], 0, 0)),
            pl.BlockSpec((1, I, H), lambda i, te, tv: (te[i], 0, 0)),
            pl.BlockSpec((1, H, I), lambda i, te, tv: (te[i], 0, 0)),
            pl.BlockSpec((TILE, 1), lambda i, te, tv: (i, 0)),
        ],
        out_specs=pl.BlockSpec((TILE, H),
                               lambda i, te, tv: (jnp.where(tv[i] == 1, i,
                                                            NT - 1), 0)),
    )
    return pl.pallas_call(
        _gmm_body,
        grid_spec=grid_spec,
        out_shape=jax.ShapeDtypeStruct((C, H), jnp.float32),
        compiler_params=pltpu.CompilerParams(
            dimension_semantics=("arbitrary",),
            vmem_limit_bytes=100 * 1024 * 1024,
        ),
    )(te, tv, xs, wg, wu, wd, ws2)


def _cgather_body(ys_hbm, stok_hbm, y8_hbm, stok_v, ybuf_v):
    cid = lax.axis_index("c")
    sid = lax.axis_index("s")
    wid = sid * 2 + cid
    base8 = wid * TPW * K

    pltpu.sync_copy(stok_hbm.at[pl.ds(base8, TPW * K)], stok_v)
    for j in range(TPW * K // 32):
        idx = stok_v.at[pl.ds(j * 32, 32)]
        pltpu.sync_copy(ys_hbm.at[idx], ybuf_v)
        pltpu.sync_copy(ybuf_v, y8_hbm.at[pl.ds(base8 + j * 32, 32)])


def _cgather(ys, stok):
    mesh = plsc.VectorSubcoreMesh(core_axis_name="c", subcore_axis_name="s")
    f = functools.partial(
        pl.kernel, _cgather_body, mesh=mesh,
        out_type=jax.ShapeDtypeStruct((B * K, H), jnp.float32),
        scratch_types=[
            pltpu.VMEM((TPW * K,), jnp.int32),
            pltpu.VMEM((32, H), jnp.float32),
        ],
        compiler_params=pltpu.CompilerParams(needs_layout_passes=False),
    )()
    return f(ys, stok)


RED_B = 128


def _reduce_body(y8_ref, out_ref):
    x = y8_ref[...].reshape(RED_B, K, H)
    out_ref[...] = jnp.sum(x, axis=1)


def _reduce(y8):
    return pl.pallas_call(
        _reduce_body,
        grid=(B // RED_B,),
        in_specs=[pl.BlockSpec((RED_B * K, H), lambda b: (b, 0))],
        out_specs=pl.BlockSpec((RED_B, H), lambda b: (b, 0)),
        out_shape=jax.ShapeDtypeStruct((B, H), jnp.float32),
    )(y8)


def kernel(hidden_states, gate_weight, w_gate_proj, w_up_proj, w_down_proj):
    ids, rank, w, offs, te, tv = _router(hidden_states, gate_weight)
    idsT = ids.T
    rankT = rank.T
    wT = w.T
    xs, ws, stok = _dispatch(hidden_states, idsT, rankT, wT,
                             offs.reshape(E))
    ys = _gmm(te.reshape(NT), tv.reshape(NT), xs, w_gate_proj, w_up_proj,
              w_down_proj, ws.reshape(C, 1))
    y8 = _cgather(ys, stok)
    return _reduce(y8)


# speedup vs baseline: 1.1073x; 1.0103x over previous
import functools

import jax
import jax.numpy as jnp
from jax import lax
from jax.experimental import pallas as pl
from jax.experimental.pallas import tpu as pltpu
from jax.experimental.pallas import tpu_sc as plsc

B, H, I, E, K = 4096, 2048, 1024, 64, 8
BLK_B = 512
TILE = 256
NT = (B * K + E * (TILE - 1) + TILE - 1) // TILE
C = NT * TILE
NW = 32
TPW = B // NW


def _router_body(x_ref, gw_ref, ids_ref, rank_ref, w_ref, offs_ref, te_ref,
                 tv_ref, carry):
    b = pl.program_id(0)

    @pl.when(b == 0)
    def _():
        carry[...] = jnp.zeros_like(carry)

    x = x_ref[...]
    logits = lax.dot_general(x, gw_ref[...], (((1,), (1,)), ((), ())),
                             preferred_element_type=jnp.float32)
    iota = lax.broadcasted_iota(jnp.int32, (BLK_B, E), 1)
    l = logits
    onehots, vals, idxs = [], [], []
    for _ in range(K):
        mx = jnp.max(l, axis=1, keepdims=True)
        idx = jnp.min(jnp.where(l == mx, iota, E), axis=1, keepdims=True)
        oh = (iota == idx)
        onehots.append(oh)
        vals.append(mx)
        idxs.append(idx)
        l = jnp.where(oh, -jnp.inf, l)
    v0 = vals[0]
    exps = [jnp.exp(v - v0) for v in vals]
    denom = exps[0]
    for ev in exps[1:]:
        denom = denom + ev

    sel = onehots[0].astype(jnp.float32)
    for oh in onehots[1:]:
        sel = sel + oh.astype(jnp.float32)

    ri = lax.broadcasted_iota(jnp.int32, (BLK_B, BLK_B), 0)
    ci = lax.broadcasted_iota(jnp.int32, (BLK_B, BLK_B), 1)
    tri = (ri > ci).astype(jnp.bfloat16)
    cum = lax.dot_general(tri, sel.astype(jnp.bfloat16),
                          (((1,), (0,)), ((), ())),
                          preferred_element_type=jnp.float32)
    posf = cum + carry[0:1, :]

    rank_cols, id_cols, w_cols = [], [], []
    for k in range(K):
        oh = onehots[k]
        rank_cols.append(jnp.sum(jnp.where(oh, posf, 0.0), axis=1,
                                 keepdims=True))
        id_cols.append(idxs[k])
        w_cols.append(exps[k] / denom)
    ids_ref[...] = jnp.concatenate(id_cols, axis=1)
    rank_ref[...] = jnp.concatenate(rank_cols, axis=1).astype(jnp.int32)
    w_ref[...] = jnp.concatenate(w_cols, axis=1)

    newc = carry[0:1, :] + jnp.sum(sel, axis=0, keepdims=True)
    carry[0:1, :] = newc

    p = jnp.floor((newc + (TILE - 1.0)) * (1.0 / TILE)) * TILE
    rie = lax.broadcasted_iota(jnp.int32, (E, E), 0)
    cie = lax.broadcasted_iota(jnp.int32, (E, E), 1)
    triu = (rie < cie).astype(jnp.bfloat16)
    offs = lax.dot_general(p.astype(jnp.bfloat16), triu,
                           (((1,), (0,)), ((), ())),
                           preferred_element_type=jnp.float32)
    offs_ref[...] = offs.astype(jnp.int32)
    ends = (offs + p) * (1.0 / TILE)
    tt = jnp.sum(p) * (1.0 / TILE)
    ti = lax.broadcasted_iota(jnp.int32, (NT, E), 0).astype(jnp.float32)
    te = jnp.sum((ti >= ends).astype(jnp.float32), axis=1, keepdims=True)
    te_ref[...] = jnp.minimum(te, E - 1).astype(jnp.int32)
    tiv = lax.broadcasted_iota(jnp.int32, (NT, 1), 0).astype(jnp.float32)
    tv_ref[...] = (tiv < tt).astype(jnp.int32)


def _router(hidden_states, gate_weight):
    return pl.pallas_call(
        _router_body,
        grid=(B // BLK_B,),
        in_specs=[
            pl.BlockSpec((BLK_B, H), lambda b: (b, 0)),
            pl.BlockSpec((E, H), lambda b: (0, 0)),
        ],
        out_specs=[
            pl.BlockSpec((BLK_B, K), lambda b: (b, 0)),
            pl.BlockSpec((BLK_B, K), lambda b: (b, 0)),
            pl.BlockSpec((BLK_B, K), lambda b: (b, 0)),
            pl.BlockSpec((1, E), lambda b: (0, 0)),
            pl.BlockSpec((NT, 1), lambda b: (0, 0)),
            pl.BlockSpec((NT, 1), lambda b: (0, 0)),
        ],
        out_shape=[
            jax.ShapeDtypeStruct((B, K), jnp.int32),
            jax.ShapeDtypeStruct((B, K), jnp.int32),
            jax.ShapeDtypeStruct((B, K), jnp.float32),
            jax.ShapeDtypeStruct((1, E), jnp.int32),
            jax.ShapeDtypeStruct((NT, 1), jnp.int32),
            jax.ShapeDtypeStruct((NT, 1), jnp.int32),
        ],
        scratch_shapes=[pltpu.VMEM((8, E), jnp.float32)],
    )(hidden_states, gate_weight)


def _dispatch_body(x_hbm, idsT_hbm, rankT_hbm, wT_hbm, offs_hbm,
                   xs_hbm, ws_hbm, stok_hbm,
                   offs_v, ids_v, rank_v, wv_v, slots_v, stok_v, xbuf_v):
    cid = lax.axis_index("c")
    sid = lax.axis_index("s")
    wid = sid * 2 + cid
    base = wid * TPW

    pltpu.sync_copy(offs_hbm, offs_v)
    pltpu.sync_copy(idsT_hbm.at[:, pl.ds(base, TPW)], ids_v)
    pltpu.sync_copy(rankT_hbm.at[:, pl.ds(base, TPW)], rank_v)
    pltpu.sync_copy(wT_hbm.at[:, pl.ds(base, TPW)], wv_v)

    iota = lax.iota(jnp.int32, 16)
    for k in range(K):
        for j in range(TPW // 16):
            e = ids_v[k, pl.ds(j * 16, 16)]
            r = rank_v[k, pl.ds(j * 16, 16)]
            slot = plsc.load_gather(offs_v, [e]) + r
            slots_v[k * (TPW // 32) + j // 2, pl.ds((j % 2) * 16, 16)] = slot
            plsc.store_scatter(stok_v, [(j * 16 + iota) * K + k], slot)
    pltpu.sync_copy(stok_v, stok_hbm.at[pl.ds(base * K, TPW * K)])

    for c in range(TPW // 32):
        pltpu.sync_copy(x_hbm.at[pl.ds(base + c * 32, 32)], xbuf_v)
        for k in range(K):
            idx = slots_v.at[k * (TPW // 32) + c]
            pltpu.sync_copy(xbuf_v, xs_hbm.at[idx])
            pltpu.sync_copy(wv_v.at[k, pl.ds(c * 32, 32)], ws_hbm.at[idx])


def _dispatch(x, idsT, rankT, wT, offs):
    mesh = plsc.VectorSubcoreMesh(core_axis_name="c", subcore_axis_name="s")
    f = functools.partial(
        pl.kernel, _dispatch_body, mesh=mesh,
        out_type=[
            jax.ShapeDtypeStruct((C, H), jnp.float32),
            jax.ShapeDtypeStruct((C,), jnp.float32),
            jax.ShapeDtypeStruct((B * K,), jnp.int32),
        ],
        scratch_types=[
            pltpu.VMEM((E,), jnp.int32),
            pltpu.VMEM((K, TPW), jnp.int32),
            pltpu.VMEM((K, TPW), jnp.int32),
            pltpu.VMEM((K, TPW), jnp.float32),
            pltpu.VMEM((K * TPW // 32, 32), jnp.int32),
            pltpu.VMEM((TPW * K,), jnp.int32),
            pltpu.VMEM((32, H), jnp.float32),
        ],
        compiler_params=pltpu.CompilerParams(needs_layout_passes=False),
    )()
    return f(x, idsT, rankT, wT, offs)


def _gmm_body(te_ref, tv_ref, xs_ref, wg_ref, wu_ref, wd_ref, w_ref, out_ref):
    i = pl.program_id(0)

    @pl.when(tv_ref[i] == 1)
    def _():
        x = xs_ref[...]
        g = lax.dot_general(x, wg_ref[0], (((1,), (1,)), ((), ())),
                            preferred_element_type=jnp.float32)
        u = lax.dot_general(x, wu_ref[0], (((1,), (1,)), ((), ())),
                            preferred_element_type=jnp.float32)
        h = g * jax.nn.sigmoid(g) * u
        eo = lax.dot_general(h, wd_ref[0], (((1,), (1,)), ((), ())),
                             preferred_element_type=jnp.float32)
        out_ref[...] = eo * w_ref[...]


def _gmm(te, tv, xs, wg, wu, wd, ws2):
    grid_spec = pltpu.PrefetchScalarGridSpec(
        num_scalar_prefetch=2,
        grid=(NT,),
        in_specs=[
            pl.BlockSpec((TILE, H),
                         lambda i, te, tv: (jnp.where(tv[i] == 1, i, NT - 1),
                                            0)),
            pl.BlockSpec((1, I, H), lambda i, te, tv: (te[i], 0, 0)),
            pl.BlockSpec((1, I, H), lambda i, te, tv: (te[i], 0, 0)),
            pl.BlockSpec((1, H, I), lambda i, te, tv: (te[i], 0, 0)),
            pl.BlockSpec((TILE, 1), lambda i, te, tv: (i, 0)),
        ],
        out_specs=pl.BlockSpec((TILE, H),
                               lambda i, te, tv: (jnp.where(tv[i] == 1, i,
                                                            NT - 1), 0)),
    )
    return pl.pallas_call(
        _gmm_body,
        grid_spec=grid_spec,
        out_shape=jax.ShapeDtypeStruct((C, H), jnp.float32),
        compiler_params=pltpu.CompilerParams(
            dimension_semantics=("arbitrary",),
            vmem_limit_bytes=100 * 1024 * 1024,
        ),
    )(te, tv, xs, wg, wu, wd, ws2)


CGC = 16


def _cgather_body(ys_hbm, stok_hbm, y8_hbm, stok_v, ybuf0, ybuf1,
                  gsem0, gsem1, wsem0, wsem1):
    cid = lax.axis_index("c")
    sid = lax.axis_index("s")
    wid = sid * 2 + cid
    base8 = wid * TPW * K

    pltpu.sync_copy(stok_hbm.at[pl.ds(base8, TPW * K)], stok_v)
    bufs = (ybuf0, ybuf1)
    gsems = (gsem0, gsem1)
    wsems = (wsem0, wsem1)
    nch = TPW * K // CGC
    gh, wh = {}, {}
    for j in range(nch):
        p = j % 2
        if j >= 2:
            wh[j - 2].wait()
        gh[j] = pltpu.async_copy(ys_hbm.at[stok_v.at[pl.ds(j * CGC, CGC)]],
                                 bufs[p], gsems[p])
        if j >= 1:
            q = (j - 1) % 2
            gh[j - 1].wait()
            wh[j - 1] = pltpu.async_copy(
                bufs[q], y8_hbm.at[pl.ds(base8 + (j - 1) * CGC, CGC)],
                wsems[q])
    p = (nch - 1) % 2
    gh[nch - 1].wait()
    wh[nch - 1] = pltpu.async_copy(
        bufs[p], y8_hbm.at[pl.ds(base8 + (nch - 1) * CGC, CGC)], wsems[p])
    wh[nch - 2].wait()
    wh[nch - 1].wait()


def _cgather(ys, stok):
    mesh = plsc.VectorSubcoreMesh(core_axis_name="c", subcore_axis_name="s")
    f = functools.partial(
        pl.kernel, _cgather_body, mesh=mesh,
        out_type=jax.ShapeDtypeStruct((B * K, H), jnp.float32),
        scratch_types=[
            pltpu.VMEM((TPW * K,), jnp.int32),
            pltpu.VMEM((CGC, H), jnp.float32),
            pltpu.VMEM((CGC, H), jnp.float32),
            pltpu.SemaphoreType.DMA,
            pltpu.SemaphoreType.DMA,
            pltpu.SemaphoreType.DMA,
            pltpu.SemaphoreType.DMA,
        ],
        compiler_params=pltpu.CompilerParams(needs_layout_passes=False),
    )()
    return f(ys, stok)


RED_B = 128


def _reduce_body(y8_ref, out_ref):
    x = y8_ref[...].reshape(RED_B, K, H)
    out_ref[...] = jnp.sum(x, axis=1)


def _reduce(y8):
    return pl.pallas_call(
        _reduce_body,
        grid=(B // RED_B,),
        in_specs=[pl.BlockSpec((RED_B * K, H), lambda b: (b, 0))],
        out_specs=pl.BlockSpec((RED_B, H), lambda b: (b, 0)),
        out_shape=jax.ShapeDtypeStruct((B, H), jnp.float32),
    )(y8)


def kernel(hidden_states, gate_weight, w_gate_proj, w_up_proj, w_down_proj):
    ids, rank, w, offs, te, tv = _router(hidden_states, gate_weight)
    idsT = ids.T
    rankT = rank.T
    wT = w.T
    xs, ws, stok = _dispatch(hidden_states, idsT, rankT, wT,
                             offs.reshape(E))
    ys = _gmm(te.reshape(NT), tv.reshape(NT), xs, w_gate_proj, w_up_proj,
              w_down_proj, ws.reshape(C, 1))
    y8 = _cgather(ys, stok)
    return _reduce(y8)


# speedup vs baseline: 1.1081x; 1.0008x over previous
import functools

import jax
import jax.numpy as jnp
from jax import lax
from jax.experimental import pallas as pl
from jax.experimental.pallas import tpu as pltpu
from jax.experimental.pallas import tpu_sc as plsc

B, H, I, E, K = 4096, 2048, 1024, 64, 8
BLK_B = 512
TILE = 256
NT = (B * K + E * (TILE - 1) + TILE - 1) // TILE
C = NT * TILE
NW = 32
TPW = B // NW


def _router_body(x_ref, gw_ref, ids_ref, rank_ref, w_ref, offs_ref, te_ref,
                 tv_ref, carry):
    b = pl.program_id(0)

    @pl.when(b == 0)
    def _():
        carry[...] = jnp.zeros_like(carry)

    x = x_ref[...]
    logits = lax.dot_general(x, gw_ref[...], (((1,), (1,)), ((), ())),
                             preferred_element_type=jnp.float32)
    iota = lax.broadcasted_iota(jnp.int32, (BLK_B, E), 1)
    l = logits
    onehots, vals, idxs = [], [], []
    for _ in range(K):
        mx = jnp.max(l, axis=1, keepdims=True)
        idx = jnp.min(jnp.where(l == mx, iota, E), axis=1, keepdims=True)
        oh = (iota == idx)
        onehots.append(oh)
        vals.append(mx)
        idxs.append(idx)
        l = jnp.where(oh, -jnp.inf, l)
    v0 = vals[0]
    exps = [jnp.exp(v - v0) for v in vals]
    denom = exps[0]
    for ev in exps[1:]:
        denom = denom + ev

    sel = onehots[0].astype(jnp.float32)
    for oh in onehots[1:]:
        sel = sel + oh.astype(jnp.float32)

    ri = lax.broadcasted_iota(jnp.int32, (BLK_B, BLK_B), 0)
    ci = lax.broadcasted_iota(jnp.int32, (BLK_B, BLK_B), 1)
    tri = (ri > ci).astype(jnp.bfloat16)
    cum = lax.dot_general(tri, sel.astype(jnp.bfloat16),
                          (((1,), (0,)), ((), ())),
                          preferred_element_type=jnp.float32)
    posf = cum + carry[0:1, :]

    rank_cols, id_cols, w_cols = [], [], []
    for k in range(K):
        oh = onehots[k]
        rank_cols.append(jnp.sum(jnp.where(oh, posf, 0.0), axis=1,
                                 keepdims=True))
        id_cols.append(idxs[k])
        w_cols.append(exps[k] / denom)
    ids_ref[...] = jnp.concatenate(id_cols, axis=1)
    rank_ref[...] = jnp.concatenate(rank_cols, axis=1).astype(jnp.int32)
    w_ref[...] = jnp.concatenate(w_cols, axis=1)

    newc = carry[0:1, :] + jnp.sum(sel, axis=0, keepdims=True)
    carry[0:1, :] = newc

    p = jnp.floor((newc + (TILE - 1.0)) * (1.0 / TILE)) * TILE
    rie = lax.broadcasted_iota(jnp.int32, (E, E), 0)
    cie = lax.broadcasted_iota(jnp.int32, (E, E), 1)
    triu = (rie < cie).astype(jnp.bfloat16)
    offs = lax.dot_general(p.astype(jnp.bfloat16), triu,
                           (((1,), (0,)), ((), ())),
                           preferred_element_type=jnp.float32)
    offs_ref[...] = offs.astype(jnp.int32)
    ends = (offs + p) * (1.0 / TILE)
    tt = jnp.sum(p) * (1.0 / TILE)
    ti = lax.broadcasted_iota(jnp.int32, (NT, E), 0).astype(jnp.float32)
    te = jnp.sum((ti >= ends).astype(jnp.float32), axis=1, keepdims=True)
    te_ref[...] = jnp.minimum(te, E - 1).astype(jnp.int32)
    tiv = lax.broadcasted_iota(jnp.int32, (NT, 1), 0).astype(jnp.float32)
    tv_ref[...] = (tiv < tt).astype(jnp.int32)


def _router(hidden_states, gate_weight):
    return pl.pallas_call(
        _router_body,
        grid=(B // BLK_B,),
        in_specs=[
            pl.BlockSpec((BLK_B, H), lambda b: (b, 0)),
            pl.BlockSpec((E, H), lambda b: (0, 0)),
        ],
        out_specs=[
            pl.BlockSpec((BLK_B, K), lambda b: (b, 0)),
            pl.BlockSpec((BLK_B, K), lambda b: (b, 0)),
            pl.BlockSpec((BLK_B, K), lambda b: (b, 0)),
            pl.BlockSpec((1, E), lambda b: (0, 0)),
            pl.BlockSpec((NT, 1), lambda b: (0, 0)),
            pl.BlockSpec((NT, 1), lambda b: (0, 0)),
        ],
        out_shape=[
            jax.ShapeDtypeStruct((B, K), jnp.int32),
            jax.ShapeDtypeStruct((B, K), jnp.int32),
            jax.ShapeDtypeStruct((B, K), jnp.float32),
            jax.ShapeDtypeStruct((1, E), jnp.int32),
            jax.ShapeDtypeStruct((NT, 1), jnp.int32),
            jax.ShapeDtypeStruct((NT, 1), jnp.int32),
        ],
        scratch_shapes=[pltpu.VMEM((8, E), jnp.float32)],
    )(hidden_states, gate_weight)


def _dispatch_body(x_hbm, idsT_hbm, rankT_hbm, wT_hbm, offs_hbm,
                   xs_hbm, ws_hbm, stok_hbm,
                   offs_v, ids_v, rank_v, wv_v, slots_v, stok_v, xbuf_v):
    cid = lax.axis_index("c")
    sid = lax.axis_index("s")
    wid = sid * 2 + cid
    base = wid * TPW

    pltpu.sync_copy(offs_hbm, offs_v)
    pltpu.sync_copy(idsT_hbm.at[:, pl.ds(base, TPW)], ids_v)
    pltpu.sync_copy(rankT_hbm.at[:, pl.ds(base, TPW)], rank_v)
    pltpu.sync_copy(wT_hbm.at[:, pl.ds(base, TPW)], wv_v)

    iota = lax.iota(jnp.int32, 16)
    for k in range(K):
        for j in range(TPW // 16):
            e = ids_v[k, pl.ds(j * 16, 16)]
            r = rank_v[k, pl.ds(j * 16, 16)]
            slot = plsc.load_gather(offs_v, [e]) + r
            slots_v[k * (TPW // 32) + j // 2, pl.ds((j % 2) * 16, 16)] = slot
            plsc.store_scatter(stok_v, [(j * 16 + iota) * K + k], slot)
    pltpu.sync_copy(stok_v, stok_hbm.at[pl.ds(base * K, TPW * K)])

    for c in range(TPW // 32):
        pltpu.sync_copy(x_hbm.at[pl.ds(base + c * 32, 32)], xbuf_v)
        for k in range(K):
            idx = slots_v.at[k * (TPW // 32) + c]
            pltpu.sync_copy(xbuf_v, xs_hbm.at[idx])
            pltpu.sync_copy(wv_v.at[k, pl.ds(c * 32, 32)], ws_hbm.at[idx])


def _dispatch(x, idsT, rankT, wT, offs):
    mesh = plsc.VectorSubcoreMesh(core_axis_name="c", subcore_axis_name="s")
    f = functools.partial(
        pl.kernel, _dispatch_body, mesh=mesh,
        out_type=[
            jax.ShapeDtypeStruct((C, H), jnp.float32),
            jax.ShapeDtypeStruct((C,), jnp.float32),
            jax.ShapeDtypeStruct((B * K,), jnp.int32),
        ],
        scratch_types=[
            pltpu.VMEM((E,), jnp.int32),
            pltpu.VMEM((K, TPW), jnp.int32),
            pltpu.VMEM((K, TPW), jnp.int32),
            pltpu.VMEM((K, TPW), jnp.float32),
            pltpu.VMEM((K * TPW // 32, 32), jnp.int32),
            pltpu.VMEM((TPW * K,), jnp.int32),
            pltpu.VMEM((32, H), jnp.float32),
        ],
        compiler_params=pltpu.CompilerParams(needs_layout_passes=False),
    )()
    return f(x, idsT, rankT, wT, offs)


def _gmm_body(te_ref, tv_ref, xs_ref, wg_ref, wu_ref, wd_ref, w_ref, out_ref):
    i = pl.program_id(0)

    @pl.when(tv_ref[i] == 1)
    def _():
        x = xs_ref[...]
        g = lax.dot_general(x, wg_ref[0], (((1,), (1,)), ((), ())),
                            preferred_element_type=jnp.float32)
        u = lax.dot_general(x, wu_ref[0], (((1,), (1,)), ((), ())),
                            preferred_element_type=jnp.float32)
        h = g * jax.nn.sigmoid(g) * u
        eo = lax.dot_general(h, wd_ref[0], (((1,), (1,)), ((), ())),
                             preferred_element_type=jnp.float32)
        out_ref[...] = eo * w_ref[...]


def _gmm(te, tv, xs, wg, wu, wd, ws2):
    grid_spec = pltpu.PrefetchScalarGridSpec(
        num_scalar_prefetch=2,
        grid=(NT,),
        in_specs=[
            pl.BlockSpec((TILE, H),
                         lambda i, te, tv: (jnp.where(tv[i] == 1, i, NT - 1),
                                            0)),
            pl.BlockSpec((1, I, H), lambda i, te, tv: (te[i], 0, 0)),
            pl.BlockSpec((1, I, H), lambda i, te, tv: (te[i], 0, 0)),
            pl.BlockSpec((1, H, I), lambda i, te, tv: (te[i], 0, 0)),
            pl.BlockSpec((TILE, 1), lambda i, te, tv: (i, 0)),
        ],
        out_specs=pl.BlockSpec((TILE, H),
                               lambda i, te, tv: (jnp.where(tv[i] == 1, i,
                                                            NT - 1), 0)),
    )
    return pl.pallas_call(
        _gmm_body,
        grid_spec=grid_spec,
        out_shape=jax.ShapeDtypeStruct((C, H), jnp.float32),
        compiler_params=pltpu.CompilerParams(
            dimension_semantics=("arbitrary",),
            vmem_limit_bytes=100 * 1024 * 1024,
        ),
    )(te, tv, xs, wg, wu, wd, ws2)


CGC = 16


def _cgather_body(ys_hbm, stok_hbm, y8_hbm, stok_v, ybuf0, ybuf1,
                  gsem0, gsem1, wsem0, wsem1):
    cid = lax.axis_index("c")
    sid = lax.axis_index("s")
    wid = sid * 2 + cid
    base8 = wid * TPW * K

    pltpu.sync_copy(stok_hbm.at[pl.ds(base8, TPW * K)], stok_v)
    bufs = (ybuf0, ybuf1)
    gsems = (gsem0, gsem1)
    wsems = (wsem0, wsem1)
    nch = TPW * K // CGC
    gh, wh = {}, {}
    for j in range(nch):
        p = j % 2
        if j >= 2:
            wh[j - 2].wait()
        gh[j] = pltpu.async_copy(ys_hbm.at[stok_v.at[pl.ds(j * CGC, CGC)]],
                                 bufs[p], gsems[p])
        if j >= 1:
            q = (j - 1) % 2
            gh[j - 1].wait()
            wh[j - 1] = pltpu.async_copy(
                bufs[q], y8_hbm.at[pl.ds(base8 + (j - 1) * CGC, CGC)],
                wsems[q])
    p = (nch - 1) % 2
    gh[nch - 1].wait()
    wh[nch - 1] = pltpu.async_copy(
        bufs[p], y8_hbm.at[pl.ds(base8 + (nch - 1) * CGC, CGC)], wsems[p])
    wh[nch - 2].wait()
    wh[nch - 1].wait()


def _cgather(ys, stok):
    mesh = plsc.VectorSubcoreMesh(core_axis_name="c", subcore_axis_name="s")
    f = functools.partial(
        pl.kernel, _cgather_body, mesh=mesh,
        out_type=jax.ShapeDtypeStruct((B * K, H), jnp.float32),
        scratch_types=[
            pltpu.VMEM((TPW * K,), jnp.int32),
            pltpu.VMEM((CGC, H), jnp.float32),
            pltpu.VMEM((CGC, H), jnp.float32),
            pltpu.SemaphoreType.DMA,
            pltpu.SemaphoreType.DMA,
            pltpu.SemaphoreType.DMA,
            pltpu.SemaphoreType.DMA,
        ],
        compiler_params=pltpu.CompilerParams(needs_layout_passes=False),
    )()
    return f(ys, stok)


RED_B = 256


def _reduce_body(y8_ref, out_ref):
    x = y8_ref[...].reshape(RED_B, K, H)
    out_ref[...] = jnp.sum(x, axis=1)


def _reduce(y8):
    return pl.pallas_call(
        _reduce_body,
        grid=(B // RED_B,),
        in_specs=[pl.BlockSpec((RED_B * K, H), lambda b: (b, 0))],
        out_specs=pl.BlockSpec((RED_B, H), lambda b: (b, 0)),
        out_shape=jax.ShapeDtypeStruct((B, H), jnp.float32),
    )(y8)


def kernel(hidden_states, gate_weight, w_gate_proj, w_up_proj, w_down_proj):
    ids, rank, w, offs, te, tv = _router(hidden_states, gate_weight)
    idsT = ids.T
    rankT = rank.T
    wT = w.T
    xs, ws, stok = _dispatch(hidden_states, idsT, rankT, wT,
                             offs.reshape(E))
    ys = _gmm(te.reshape(NT), tv.reshape(NT), xs, w_gate_proj, w_up_proj,
              w_down_proj, ws.reshape(C, 1))
    y8 = _cgather(ys, stok)
    return _reduce(y8)
